# baseline (speedup 1.0000x reference)
import functools

import jax
import jax.numpy as jnp
from jax import lax
from jax.experimental import pallas as pl
from jax.experimental.pallas import tpu as pltpu
from jax.experimental.pallas import tpu_sc as plsc

B = 4096
S = 200
D = 64
NW = 32
BC = B // NW
NBUF = 4
LOOKAHEAD = 2
NQ = D // 16


def _embedder(idx_hbm, tok_hbm, pos_hbm, out_hbm, idx_v, pos_v, buf, buft, gsem, osem):
    wid = lax.axis_index("s") * 2 + lax.axis_index("c")
    b0 = wid * BC
    pltpu.sync_copy(pos_hbm, pos_v)
    pltpu.sync_copy(idx_hbm.at[:, pl.ds(b0, BC)], idx_v)

    def start_gather(i, slot):
        pltpu.async_copy(tok_hbm.at[idx_v.at[i]], buf.at[slot], gsem.at[slot])

    def start_store(i, slot):
        pltpu.async_copy(buft.at[slot], out_hbm.at[i, :, wid], osem.at[slot])

    def wait_gather(i, slot):
        pltpu.make_async_copy(tok_hbm.at[idx_v.at[i]], buf.at[slot], gsem.at[slot]).wait()

    def wait_store(i, slot):
        pltpu.make_async_copy(buft.at[slot], out_hbm.at[i, :, wid], osem.at[slot]).wait()

    for j in range(LOOKAHEAD):
        start_gather(j, j)

    row16 = jnp.arange(16, dtype=jnp.int32)
    dbx = [(row16 + q * 16) // 8 for q in range(NQ)]
    drx = [(row16 + q * 16) % 8 for q in range(NQ)]

    def group(g, carry):
        for slot in range(NBUF):
            i = g * NBUF + slot
            j = i + LOOKAHEAD
            slot_j = (slot + LOOKAHEAD) % NBUF

            @pl.when(j < S)
            def _prefetch():
                @pl.when(j >= NBUF)
                def _drain():
                    wait_store(j - NBUF, slot_j)

                start_gather(j, slot_j)

            wait_gather(i, slot)

            pos_q = [pos_v[i, pl.ds(q * 16, 16)] for q in range(NQ)]

            def do_b(b, _slot=slot, _pos=pos_q):
                bspl = jnp.full((16,), b, dtype=jnp.int32)
                for q in range(NQ):
                    v = buf[_slot, b, pl.ds(q * 16, 16)]
                    plsc.store_scatter(
                        buft.at[_slot], [dbx[q], drx[q], bspl], v + _pos[q]
                    )

            plsc.parallel_loop(0, BC, 1, unroll=8)(do_b)
            start_store(i, slot)
        return carry

    lax.fori_loop(0, S // NBUF, group, 0)

    for i in range(S - NBUF, S):
        wait_store(i, i % NBUF)


@jax.jit
def _run(idx_t, token_table, pos_table):
    kern = pl.kernel(
        _embedder,
        out_type=jax.ShapeDtypeStruct((S, D // 8, NW, 8, BC), jnp.float32),
        mesh=plsc.VectorSubcoreMesh(core_axis_name="c", subcore_axis_name="s"),
        scratch_types=[
            pltpu.VMEM((S, BC), jnp.int32),
            pltpu.VMEM((S, D), jnp.float32),
            pltpu.VMEM((NBUF, BC, D), jnp.float32),
            pltpu.VMEM((NBUF, D // 8, 8, BC), jnp.float32),
            pltpu.SemaphoreType.DMA((NBUF,)),
            pltpu.SemaphoreType.DMA((NBUF,)),
        ],
        compiler_params=pltpu.CompilerParams(
            use_tc_tiling_on_sc=False, needs_layout_passes=False
        ),
    )
    out5 = kern(idx_t, token_table, pos_table)
    return out5.transpose(2, 4, 0, 1, 3).reshape(B, S, D)


def kernel(input, token_table, pos_table):
    idx_t = jnp.maximum(input.T.astype(jnp.int32), 0)
    return _run(idx_t, token_table, pos_table)


# speedup vs baseline: 1.8109x; 1.8109x over previous
import functools

import jax
import jax.numpy as jnp
from jax import lax
from jax.experimental import pallas as pl
from jax.experimental.pallas import tpu as pltpu
from jax.experimental.pallas import tpu_sc as plsc

B = 4096
S = 200
D = 64
NW = 32
BC = B // NW
NBUF = 4
LOOKAHEAD = 2
NQ = D // 16


def _embedder(idx_hbm, tok_hbm, pos_hbm, out_hbm, idx_v, pos_v, buf, buft, gsem, osem):
    wid = lax.axis_index("s") * 2 + lax.axis_index("c")
    b0 = wid * BC
    pltpu.sync_copy(pos_hbm, pos_v)
    pltpu.sync_copy(idx_hbm.at[:, pl.ds(b0, BC)], idx_v)

    def start_gather(i, slot):
        pltpu.async_copy(tok_hbm.at[idx_v.at[i]], buf.at[slot], gsem.at[slot])

    def start_store(i, slot):
        pltpu.async_copy(
            buft.at[slot, :, :, pl.ds(0, BC)], out_hbm.at[i, :, wid], osem.at[slot]
        )

    def wait_gather(i, slot):
        pltpu.make_async_copy(tok_hbm.at[idx_v.at[i]], buf.at[slot], gsem.at[slot]).wait()

    def wait_store(i, slot):
        pltpu.make_async_copy(
            buft.at[slot, :, :, pl.ds(0, BC)], out_hbm.at[i, :, wid], osem.at[slot]
        ).wait()

    for j in range(LOOKAHEAD):
        start_gather(j, j)

    row16 = jnp.arange(16, dtype=jnp.int32)
    dbx = [(row16 + q * 16) // 8 for q in range(NQ)]
    drx = [(row16 + q * 16) % 8 for q in range(NQ)]

    def group(g, carry):
        for slot in range(NBUF):
            i = g * NBUF + slot
            j = i + LOOKAHEAD
            slot_j = (slot + LOOKAHEAD) % NBUF

            @pl.when(j < S)
            def _prefetch():
                @pl.when(j >= NBUF)
                def _drain():
                    wait_store(j - NBUF, slot_j)

                start_gather(j, slot_j)

            wait_gather(i, slot)

            pos_q = [pos_v[i, pl.ds(q * 16, 16)] for q in range(NQ)]

            def do_b(b, _slot=slot, _pos=pos_q):
                bspl = jnp.full((16,), b, dtype=jnp.int32)
                for q in range(NQ):
                    v = buf[_slot, b, pl.ds(q * 16, 16)]
                    plsc.store_scatter(
                        buft.at[_slot], [dbx[q], drx[q], bspl], v + _pos[q]
                    )

            plsc.parallel_loop(0, BC, 1, unroll=4)(do_b)
            start_store(i, slot)
        return carry

    lax.fori_loop(0, S // NBUF, group, 0)

    for i in range(S - NBUF, S):
        wait_store(i, i % NBUF)


@jax.jit
def _run(idx_t, token_table, pos_table):
    kern = pl.kernel(
        _embedder,
        out_type=jax.ShapeDtypeStruct((S, D // 8, NW, 8, BC), jnp.float32),
        mesh=plsc.VectorSubcoreMesh(core_axis_name="c", subcore_axis_name="s"),
        scratch_types=[
            pltpu.VMEM((S, BC), jnp.int32),
            pltpu.VMEM((S, D), jnp.float32),
            pltpu.VMEM((NBUF, BC, D), jnp.float32),
            pltpu.VMEM((NBUF, D // 8, 8, BC + 1), jnp.float32),
            pltpu.SemaphoreType.DMA((NBUF,)),
            pltpu.SemaphoreType.DMA((NBUF,)),
        ],
        compiler_params=pltpu.CompilerParams(
            use_tc_tiling_on_sc=False, needs_layout_passes=False
        ),
    )
    out5 = kern(idx_t, token_table, pos_table)
    return out5.transpose(2, 4, 0, 1, 3).reshape(B, S, D)


def kernel(input, token_table, pos_table):
    idx_t = jnp.maximum(input.T.astype(jnp.int32), 0)
    return _run(idx_t, token_table, pos_table)
